# Initial kernel scaffold; baseline (speedup 1.0000x reference)
import functools

import jax
import jax.numpy as jnp
from jax import lax
from jax.experimental import pallas as pl
from jax.experimental.pallas import tpu as pltpu
from jax.experimental.pallas import tpu_sc as plsc

VOCAB = 1000
D = 1000
BATCH = 4096
SEQ = 50
B = BATCH * SEQ
NW = 32
B_PER_W = B // NW
CHUNK = 50
N_CHUNKS = B_PER_W // CHUNK

_mesh = plsc.VectorSubcoreMesh(core_axis_name="c", subcore_axis_name="s")


@functools.partial(
    pl.kernel,
    mesh=_mesh,
    out_type=jax.ShapeDtypeStruct((B, D), jnp.float32),
    scratch_types=[
        pltpu.VMEM((N_CHUNKS, CHUNK), jnp.int32),
        pltpu.VMEM((CHUNK, D), jnp.float32),
        pltpu.SemaphoreType.DMA,
    ],
)
def _gather_rows(idx_hbm, table_hbm, out_hbm, idx_v, rows_v, gsem):
    wid = lax.axis_index("s") * 2 + lax.axis_index("c")
    base = wid * B_PER_W
    pltpu.sync_copy(idx_hbm.at[wid], idx_v)

    def body(c, carry):
        pltpu.async_copy(table_hbm.at[idx_v.at[c]], rows_v, gsem).wait()
        pltpu.sync_copy(rows_v, out_hbm.at[pl.ds(base + c * CHUNK, CHUNK)])
        return carry

    lax.fori_loop(0, N_CHUNKS, body, 0)


def kernel(idx, table):
    idx_flat = idx.reshape(NW, N_CHUNKS, CHUNK).astype(jnp.int32)
    out = _gather_rows(idx_flat, table)
    return out.reshape(BATCH, SEQ, D)


# speedup vs baseline: 1.0207x; 1.0207x over previous
import functools

import jax
import jax.numpy as jnp
from jax import lax
from jax.experimental import pallas as pl
from jax.experimental.pallas import tpu as pltpu
from jax.experimental.pallas import tpu_sc as plsc

VOCAB = 1000
D = 1000
BATCH = 4096
SEQ = 50
B = BATCH * SEQ
NW = 32
B_PER_W = B // NW
CHUNK = 64
N_CHUNKS = B_PER_W // CHUNK

_mesh = plsc.VectorSubcoreMesh(core_axis_name="c", subcore_axis_name="s")


@functools.partial(
    pl.kernel,
    mesh=_mesh,
    out_type=jax.ShapeDtypeStruct((B, D), jnp.float32),
    compiler_params=pltpu.CompilerParams(use_tc_tiling_on_sc=False),
    scratch_types=[
        pltpu.VMEM((N_CHUNKS, CHUNK), jnp.int32),
        pltpu.VMEM((CHUNK, D), jnp.float32),
        pltpu.SemaphoreType.DMA,
    ],
)
def _gather_rows(idx_hbm, table_hbm, out_hbm, idx_v, rows_v, gsem):
    wid = lax.axis_index("s") * 2 + lax.axis_index("c")
    base = wid * B_PER_W
    pltpu.sync_copy(idx_hbm.at[wid], idx_v)

    def body(c, carry):
        pltpu.async_copy(table_hbm.at[idx_v.at[c]], rows_v, gsem).wait()
        pltpu.sync_copy(rows_v, out_hbm.at[pl.ds(base + c * CHUNK, CHUNK)])
        return carry

    lax.fori_loop(0, N_CHUNKS, body, 0)


def kernel(idx, table):
    idx_flat = idx.reshape(NW, N_CHUNKS, CHUNK).astype(jnp.int32)
    out = _gather_rows(idx_flat, table)
    return out.reshape(BATCH, SEQ, D)


# speedup vs baseline: 1.0319x; 1.0110x over previous
import functools

import jax
import jax.numpy as jnp
from jax import lax
from jax.experimental import pallas as pl
from jax.experimental.pallas import tpu as pltpu
from jax.experimental.pallas import tpu_sc as plsc

VOCAB = 1000
D = 1000
BATCH = 4096
SEQ = 50
B = BATCH * SEQ
NW = 32
B_PER_W = B // NW
CHUNK = 40
N_CHUNKS = B_PER_W // CHUNK
N_PAIRS = N_CHUNKS // 2

_mesh = plsc.VectorSubcoreMesh(core_axis_name="c", subcore_axis_name="s")


@functools.partial(
    pl.kernel,
    mesh=_mesh,
    out_type=jax.ShapeDtypeStruct((B, D), jnp.float32),
    compiler_params=pltpu.CompilerParams(use_tc_tiling_on_sc=False),
    scratch_types=[
        pltpu.VMEM((N_CHUNKS, CHUNK), jnp.int32),
        pltpu.VMEM((CHUNK, D), jnp.float32),
        pltpu.VMEM((CHUNK, D), jnp.float32),
        pltpu.SemaphoreType.DMA,
        pltpu.SemaphoreType.DMA,
        pltpu.SemaphoreType.DMA,
        pltpu.SemaphoreType.DMA,
    ],
)
def _gather_rows(idx_hbm, table_hbm, out_hbm, idx_v, rows0, rows1,
                 g0, g1, s0, s1):
    wid = lax.axis_index("s") * 2 + lax.axis_index("c")
    base = wid * B_PER_W
    pltpu.sync_copy(idx_hbm.at[wid], idx_v)

    def gather(c, buf, sem):
        return pltpu.async_copy(table_hbm.at[idx_v.at[c]], buf, sem)

    def scatter(c, buf, sem):
        return pltpu.async_copy(buf, out_hbm.at[pl.ds(base + c * CHUNK, CHUNK)], sem)

    def wait_scatter(buf, sem):
        pltpu.make_async_copy(buf, out_hbm.at[pl.ds(base, CHUNK)], sem).wait()

    gather(0, rows0, g0)

    def body(i, carry):
        c0 = 2 * i

        @pl.when(i > 0)
        def _():
            wait_scatter(rows1, s1)
        gather(c0 + 1, rows1, g1)
        pltpu.make_async_copy(table_hbm.at[idx_v.at[c0]], rows0, g0).wait()
        scatter(c0, rows0, s0)

        pltpu.make_async_copy(table_hbm.at[idx_v.at[c0]], rows1, g1).wait()

        @pl.when(i < N_PAIRS - 1)
        def _():
            wait_scatter(rows0, s0)
            gather(c0 + 2, rows0, g0)
        scatter(c0 + 1, rows1, s1)
        return carry

    lax.fori_loop(0, N_PAIRS, body, 0)
    wait_scatter(rows0, s0)
    wait_scatter(rows1, s1)


def kernel(idx, table):
    idx_flat = idx.reshape(NW, N_CHUNKS, CHUNK).astype(jnp.int32)
    out = _gather_rows(idx_flat, table)
    return out.reshape(BATCH, SEQ, D)


# speedup vs baseline: 1.0357x; 1.0036x over previous
import functools

import jax
import jax.numpy as jnp
from jax import lax
from jax.experimental import pallas as pl
from jax.experimental.pallas import tpu as pltpu
from jax.experimental.pallas import tpu_sc as plsc

VOCAB = 1000
D = 1000
BATCH = 4096
SEQ = 50
NW = 32
BAT_PER_W = BATCH // NW
N_PAIRS = BAT_PER_W // 2

_mesh = plsc.VectorSubcoreMesh(core_axis_name="c", subcore_axis_name="s")


@functools.partial(
    pl.kernel,
    mesh=_mesh,
    out_type=jax.ShapeDtypeStruct((BATCH, SEQ, D), jnp.float32),
    compiler_params=pltpu.CompilerParams(use_tc_tiling_on_sc=False),
    scratch_types=[
        pltpu.VMEM((BAT_PER_W, SEQ), jnp.int32),
        pltpu.VMEM((SEQ, D), jnp.float32),
        pltpu.VMEM((SEQ, D), jnp.float32),
        pltpu.SemaphoreType.DMA,
        pltpu.SemaphoreType.DMA,
        pltpu.SemaphoreType.DMA,
        pltpu.SemaphoreType.DMA,
    ],
)
def _gather_rows(idx_hbm, table_hbm, out_hbm, idx_v, rows0, rows1,
                 g0, g1, s0, s1):
    wid = lax.axis_index("s") * 2 + lax.axis_index("c")
    base = wid * BAT_PER_W
    pltpu.sync_copy(idx_hbm.at[wid], idx_v)

    def gather(b, buf, sem):
        return pltpu.async_copy(table_hbm.at[idx_v.at[b]], buf, sem)

    def scatter(b, buf, sem):
        return pltpu.async_copy(buf, out_hbm.at[base + b], sem)

    def wait_gather(buf, sem):
        pltpu.make_async_copy(table_hbm.at[idx_v.at[0]], buf, sem).wait()

    def wait_scatter(buf, sem):
        pltpu.make_async_copy(buf, out_hbm.at[base], sem).wait()

    gather(0, rows0, g0)

    def body(i, carry):
        b0 = 2 * i

        @pl.when(i > 0)
        def _():
            wait_scatter(rows1, s1)
        gather(b0 + 1, rows1, g1)
        wait_gather(rows0, g0)
        scatter(b0, rows0, s0)

        wait_gather(rows1, g1)

        @pl.when(i < N_PAIRS - 1)
        def _():
            wait_scatter(rows0, s0)
            gather(b0 + 2, rows0, g0)
        scatter(b0 + 1, rows1, s1)
        return carry

    lax.fori_loop(0, N_PAIRS, body, 0)
    wait_scatter(rows0, s0)
    wait_scatter(rows1, s1)


def kernel(idx, table):
    idx_w = idx.reshape(NW, BAT_PER_W, SEQ).astype(jnp.int32)
    return _gather_rows(idx_w, table)


# speedup vs baseline: 2.1293x; 2.0559x over previous
import functools

import jax
import jax.numpy as jnp
from jax import lax
from jax.experimental import pallas as pl
from jax.experimental.pallas import tpu as pltpu
from jax.experimental.pallas import tpu_sc as plsc

VOCAB = 1000
D = 1000
BATCH = 4096
SEQ = 50
SEQ_PAD = 56
NW = 32
BAT_PER_W = BATCH // NW
N_PAIRS = BAT_PER_W // 2
N_U = 8
U_TAIL = D - 7 * 128

_mesh = plsc.VectorSubcoreMesh(core_axis_name="c", subcore_axis_name="s")


@functools.partial(
    pl.kernel,
    mesh=_mesh,
    out_type=(jax.ShapeDtypeStruct((BATCH, SEQ, D), jnp.float32),
              jax.ShapeDtypeStruct((BATCH, SEQ, 128), jnp.float32)),
    compiler_params=pltpu.CompilerParams(use_tc_tiling_on_sc=True),
    scratch_types=[
        pltpu.VMEM((BAT_PER_W * SEQ_PAD,), jnp.int32),
        pltpu.VMEM((2, N_U, SEQ, 128), jnp.float32),
        pltpu.SemaphoreType.DMA,
        pltpu.SemaphoreType.DMA,
        pltpu.SemaphoreType.DMA,
        pltpu.SemaphoreType.DMA,
    ],
)
def _gather_rows(idx_hbm, t0, t1, t2, t3, t4, t5, t6, t7, out_hbm, tail_hbm,
                 idx_v, bufs, g0, g1, s0, s1):
    tables = [t0, t1, t2, t3, t4, t5, t6, t7]
    gsems = [g0, g1]
    ssems = [s0, s1]
    wid = lax.axis_index("s") * 2 + lax.axis_index("c")
    slab = BAT_PER_W * SEQ_PAD
    base_b = wid * BAT_PER_W
    pltpu.sync_copy(idx_hbm.at[pl.ds(wid * slab, slab)], idx_v)

    def gather(b, d):
        ii = idx_v.at[pl.ds(b * SEQ_PAD, SEQ)]
        for u in range(N_U):
            pltpu.async_copy(tables[u].at[ii], bufs.at[d, u], gsems[d])

    def wait_gathers(d):
        ii = idx_v.at[pl.ds(0, SEQ)]
        for u in range(N_U):
            pltpu.make_async_copy(tables[u].at[ii], bufs.at[d, u], gsems[d]).wait()

    def scatter(b, d):
        gb = base_b + b
        for u in range(N_U - 1):
            pltpu.async_copy(bufs.at[d, u],
                             out_hbm.at[gb, :, pl.ds(128 * u, 128)], ssems[d])
        pltpu.async_copy(bufs.at[d, N_U - 1], tail_hbm.at[gb], ssems[d])

    def wait_scatters(d):
        for u in range(N_U - 1):
            pltpu.make_async_copy(bufs.at[d, u],
                                  out_hbm.at[base_b, :, pl.ds(128 * u, 128)],
                                  ssems[d]).wait()
        pltpu.make_async_copy(bufs.at[d, N_U - 1], tail_hbm.at[base_b],
                              ssems[d]).wait()

    gather(0, 0)

    def body(i, carry):
        b0 = 2 * i

        @pl.when(i > 0)
        def _():
            wait_scatters(1)
        gather(b0 + 1, 1)
        wait_gathers(0)
        scatter(b0, 0)

        wait_gathers(1)

        @pl.when(i < N_PAIRS - 1)
        def _():
            wait_scatters(0)
            gather(b0 + 2, 0)
        scatter(b0 + 1, 1)
        return carry

    lax.fori_loop(0, N_PAIRS, body, 0)
    wait_scatters(0)
    wait_scatters(1)


def kernel(idx, table):
    idx_p = jnp.pad(idx.astype(jnp.int32), ((0, 0), (0, SEQ_PAD - SEQ)))
    idx_flat = idx_p.reshape(-1)
    blocks = [table[:, 128 * u:128 * (u + 1)] for u in range(N_U - 1)]
    tail_block = jnp.pad(table[:, 128 * (N_U - 1):], ((0, 0), (0, 128 - U_TAIL)))
    out, tail = _gather_rows(idx_flat, *blocks, tail_block)
    return lax.dynamic_update_slice(out, tail[:, :, :U_TAIL], (0, 0, 128 * (N_U - 1)))
